# baseline (speedup 1.0000x reference)
import functools

import jax
import jax.numpy as jnp
from jax import lax
from jax.experimental import pallas as pl
from jax.experimental.pallas import tpu as pltpu
from jax.experimental.pallas import tpu_sc as plsc

_B = 4
_S = 2048
_D = 128

_info = plsc.get_sparse_core_info()
_NC = _info.num_cores
_NS = _info.num_subcores
_NW = _NC * _NS
_SPW = _S // _NW
_LANES = 16
_CHUNKS = _D // _LANES


def _body(x_hbm, tok_hbm, pos_hbm, out_hbm, idx_v, rows_v, pos_v,
          psem, isem, gsem, wsem):
    wid = lax.axis_index("s") * _NC + lax.axis_index("c")
    s0 = wid * _SPW

    with jax.named_scope("idx_stage"):
        ihandles = [
            pltpu.async_copy(x_hbm.at[b, pl.ds(s0, _SPW)], idx_v.at[b], isem)
            for b in range(_B)
        ]
        ph = pltpu.async_copy(pos_hbm.at[pl.ds(s0, _SPW)], pos_v, psem)
        for h in ihandles:
            h.wait()

    with jax.named_scope("gather_fire"):
        ghandles = [
            pltpu.async_copy(tok_hbm.at[idx_v.at[b]], rows_v.at[b], gsem)
            for b in range(_B)
        ]
    ph.wait()

    whandles = []
    for b in range(_B):
        with jax.named_scope("gwait"):
            ghandles[b].wait()

        with jax.named_scope("add"):

            @plsc.parallel_loop(0, _SPW, unroll=4)
            def _add_row(r):
                for c in range(_CHUNKS):
                    sl = pl.ds(c * _LANES, _LANES)
                    plsc.addupdate(rows_v.at[b, r, sl], pos_v[r, sl])

        whandles.append(
            pltpu.async_copy(
                rows_v.at[b],
                out_hbm.at[b, pl.ds(s0, _SPW)],
                wsem,
            )
        )
    with jax.named_scope("drain"):
        for h in whandles:
            h.wait()


@jax.jit
def _embed(x, token_table, pos_table):
    mesh = plsc.VectorSubcoreMesh(core_axis_name="c", subcore_axis_name="s")
    k = functools.partial(
        pl.kernel,
        mesh=mesh,
        out_type=jax.ShapeDtypeStruct((_B, _S, _D), jnp.float32),
        scratch_types=[
            pltpu.VMEM((_B, _SPW), jnp.int32),
            pltpu.VMEM((_B, _SPW, _D), jnp.float32),
            pltpu.VMEM((_SPW, _D), jnp.float32),
            pltpu.SemaphoreType.DMA,
            pltpu.SemaphoreType.DMA,
            pltpu.SemaphoreType.DMA,
            pltpu.SemaphoreType.DMA,
        ],
    )(_body)
    return k(x, token_table, pos_table)


def kernel(x, token_table, pos_table):
    return _embed(x.astype(jnp.int32), token_table, pos_table)
